# baseline (speedup 1.0000x reference)
import functools

import jax
import jax.numpy as jnp
from jax.experimental import pallas as pl
from jax.experimental.pallas import tpu as pltpu

B, S, C, D, TOPN = 16, 32, 4096, 256, 8


def _copy(ctx_hbm_ref, ctx_buf, sems, src_idx, slot):
    return pltpu.make_async_copy(
        ctx_hbm_ref.at[src_idx], ctx_buf.at[slot], sems.at[slot])


def _topk_kernel(q_ref, ctx_hbm_ref, w_ref, b_ref, dist_out_ref, idx_out_ref,
                 ctx_buf, sems):
    i = pl.program_id(0)
    slot = jax.lax.rem(i, 2)
    nslot = jax.lax.rem(i + 1, 2)

    @pl.when(i == 0)
    def _():
        _copy(ctx_hbm_ref, ctx_buf, sems, 0, 0).start()

    @pl.when(i + 1 < B)
    def _():
        _copy(ctx_hbm_ref, ctx_buf, sems, i + 1, nslot).start()

    _copy(ctx_hbm_ref, ctx_buf, sems, i, slot).wait()

    q = q_ref[0]
    w = w_ref[...]
    bias = b_ref[...]
    qp = jax.lax.dot_general(q, w, (((1,), (1,)), ((), ()))) + bias

    ctx = ctx_buf[slot]
    a2 = jnp.sum(qp * qp, axis=-1, keepdims=True)
    b2 = jnp.sum(ctx * ctx, axis=-1)
    ab = jax.lax.dot_general(qp, ctx, (((1,), (1,)), ((), ())))
    d2 = jnp.maximum(a2 + b2[None, :] - 2.0 * ab, 0.0)
    dist = jnp.sqrt(d2)

    iota = jax.lax.broadcasted_iota(jnp.int32, (S, C), 1)
    vals = dist
    top_vals = []
    top_idx = []
    for _ in range(TOPN):
        mv = jnp.min(vals, axis=1, keepdims=True)
        mi = jnp.argmin(vals, axis=1, keepdims=True).astype(jnp.int32)
        top_vals.append(mv)
        top_idx.append(mi)
        vals = jnp.where(iota == mi, jnp.float32(jnp.inf), vals)
    dist_out_ref[0] = jnp.concatenate(top_vals, axis=1)
    idx_out_ref[0] = jnp.concatenate(top_idx, axis=1)


@jax.jit
def kernel(query_embeddings, context_embeddings, W, b):
    bias2d = b.reshape(1, D)
    grid = (B,)
    out_dist, out_idx = pl.pallas_call(
        _topk_kernel,
        grid=grid,
        in_specs=[
            pl.BlockSpec((1, S, D), lambda i: (i, 0, 0)),
            pl.BlockSpec(memory_space=pltpu.MemorySpace.HBM),
            pl.BlockSpec((D, D), lambda i: (0, 0)),
            pl.BlockSpec((1, D), lambda i: (0, 0)),
        ],
        out_specs=[
            pl.BlockSpec((1, S, TOPN), lambda i: (i, 0, 0)),
            pl.BlockSpec((1, S, TOPN), lambda i: (i, 0, 0)),
        ],
        out_shape=[
            jax.ShapeDtypeStruct((B, S, TOPN), jnp.float32),
            jax.ShapeDtypeStruct((B, S, TOPN), jnp.int32),
        ],
        scratch_shapes=[
            pltpu.VMEM((2, C, D), jnp.float32),
            pltpu.SemaphoreType.DMA((2,)),
        ],
    )(query_embeddings, context_embeddings, W, bias2d)
    return (out_dist, out_idx)


# speedup vs baseline: 1.1169x; 1.1169x over previous
import functools

import jax
import jax.numpy as jnp
from jax.experimental import pallas as pl
from jax.experimental.pallas import tpu as pltpu

B, S, C, D, TOPN = 16, 32, 4096, 256, 8
BB = 2


def _one_batch(q, ctx, w, bias, dist_out_ref, idx_out_ref, j):
    qp = jax.lax.dot_general(q, w, (((1,), (1,)), ((), ()))) + bias

    a2 = jnp.sum(qp * qp, axis=-1, keepdims=True)
    b2 = jnp.sum(ctx * ctx, axis=-1)
    ab = jax.lax.dot_general(qp, ctx, (((1,), (1,)), ((), ())))
    d2 = jnp.maximum(a2 + b2[None, :] - 2.0 * ab, 0.0)
    dist = jnp.sqrt(d2)

    iota = jax.lax.broadcasted_iota(jnp.int32, (S, C), 1)
    inf = jnp.float32(jnp.inf)
    vals = dist
    top_vals = []
    top_idx = []
    for _ in range(TOPN):
        mi = jnp.argmin(vals, axis=1, keepdims=True).astype(jnp.int32)
        eqm = iota == mi
        mv = jnp.min(jnp.where(eqm, vals, inf), axis=1, keepdims=True)
        top_vals.append(mv)
        top_idx.append(mi)
        vals = jnp.where(eqm, inf, vals)
    dist_out_ref[j] = jnp.concatenate(top_vals, axis=1)
    idx_out_ref[j] = jnp.concatenate(top_idx, axis=1)


def _topk_kernel(q_ref, ctx_ref, w_ref, b_ref, dist_out_ref, idx_out_ref):
    w = w_ref[...]
    bias = b_ref[...]
    for j in range(BB):
        _one_batch(q_ref[j], ctx_ref[j], w, bias, dist_out_ref, idx_out_ref, j)


@jax.jit
def kernel(query_embeddings, context_embeddings, W, b):
    bias2d = b.reshape(1, D)
    grid = (B // BB,)
    out_dist, out_idx = pl.pallas_call(
        _topk_kernel,
        grid=grid,
        in_specs=[
            pl.BlockSpec((BB, S, D), lambda i: (i, 0, 0)),
            pl.BlockSpec((BB, C, D), lambda i: (i, 0, 0)),
            pl.BlockSpec((D, D), lambda i: (0, 0)),
            pl.BlockSpec((1, D), lambda i: (0, 0)),
        ],
        out_specs=[
            pl.BlockSpec((BB, S, TOPN), lambda i: (i, 0, 0)),
            pl.BlockSpec((BB, S, TOPN), lambda i: (i, 0, 0)),
        ],
        out_shape=[
            jax.ShapeDtypeStruct((B, S, TOPN), jnp.float32),
            jax.ShapeDtypeStruct((B, S, TOPN), jnp.int32),
        ],
    )(query_embeddings, context_embeddings, W, bias2d)
    return (out_dist, out_idx)


# speedup vs baseline: 1.1995x; 1.0740x over previous
import functools

import jax
import jax.numpy as jnp
from jax.experimental import pallas as pl
from jax.experimental.pallas import tpu as pltpu

B, S, C, D, TOPN = 16, 32, 4096, 256, 8
BB = 2


def _one_batch(q, ctx, w, bias, dist_out_ref, idx_out_ref, j):
    qp = jax.lax.dot_general(q, w, (((1,), (1,)), ((), ()))) + bias

    a2 = jnp.sum(qp * qp, axis=-1, keepdims=True)
    b2 = jnp.sum(ctx * ctx, axis=-1)
    ab = jax.lax.dot_general(qp, ctx, (((1,), (1,)), ((), ())))
    d2 = jnp.maximum(a2 + b2[None, :] - 2.0 * ab, 0.0)
    dist = jnp.sqrt(d2)

    iota = jax.lax.broadcasted_iota(jnp.int32, (S, C), 1)
    inf = jnp.float32(jnp.inf)
    vals = dist
    top_vals = []
    top_idx = []
    for _ in range(TOPN):
        mv = jnp.min(vals, axis=1, keepdims=True)
        mi = jnp.argmin(vals, axis=1, keepdims=True).astype(jnp.int32)
        top_vals.append(mv)
        top_idx.append(mi)
        vals = jnp.where(iota == mi, inf, vals)
    dist_out_ref[j] = jnp.concatenate(top_vals, axis=1)
    idx_out_ref[j] = jnp.concatenate(top_idx, axis=1)


def _topk_kernel(q_ref, ctx_ref, w_ref, b_ref, dist_out_ref, idx_out_ref):
    w = w_ref[...]
    bias = b_ref[...]
    for j in range(BB):
        _one_batch(q_ref[j], ctx_ref[j], w, bias, dist_out_ref, idx_out_ref, j)


@jax.jit
def kernel(query_embeddings, context_embeddings, W, b):
    bias2d = b.reshape(1, D)
    grid = (B // BB,)
    out_dist, out_idx = pl.pallas_call(
        _topk_kernel,
        grid=grid,
        in_specs=[
            pl.BlockSpec((BB, S, D), lambda i: (i, 0, 0)),
            pl.BlockSpec((BB, C, D), lambda i: (i, 0, 0)),
            pl.BlockSpec((D, D), lambda i: (0, 0)),
            pl.BlockSpec((1, D), lambda i: (0, 0)),
        ],
        out_specs=[
            pl.BlockSpec((BB, S, TOPN), lambda i: (i, 0, 0)),
            pl.BlockSpec((BB, S, TOPN), lambda i: (i, 0, 0)),
        ],
        out_shape=[
            jax.ShapeDtypeStruct((B, S, TOPN), jnp.float32),
            jax.ShapeDtypeStruct((B, S, TOPN), jnp.int32),
        ],
    )(query_embeddings, context_embeddings, W, bias2d)
    return (out_dist, out_idx)
